# baseline (speedup 1.0000x reference)
import dataclasses

import jax
import jax.numpy as jnp
from jax import lax
from jax.experimental import pallas as pl
from jax.experimental.pallas import tpu as pltpu
from jax.experimental.pallas import tpu_sc as plsc

B = 4096
L = 200
H = 128
H_ATTN = 128
MAX_LEN = 200
N = B * L
EPS = 1e-5

GATHER_WINDOW = 128
TC_BLOCK = 2048
K_CHUNKS = 4
NC = N // K_CHUNKS
LANES = 16


def _sc_gather_add(token_table, pos_table, ids, pids):
    n = ids.shape[1]
    mesh = plsc.VectorSubcoreMesh(core_axis_name="core", subcore_axis_name="subcore")

    cp = pltpu.CompilerParams()
    if "needs_layout_passes" in pltpu.CompilerParams.__dataclass_fields__:
        cp = dataclasses.replace(cp, needs_layout_passes=False)

    @pl.kernel(
        out_type=jax.ShapeDtypeStruct((n, H), jnp.float32),
        mesh=mesh,
        scratch_types=[pltpu.VMEM((MAX_LEN, H), jnp.float32)],
        compiler_params=cp,
    )
    def gather_kernel(tok_hbm, ptab_hbm, i_hbm, p_hbm, o_hbm, ptab_vmem):
        pltpu.sync_copy(ptab_hbm, ptab_vmem)
        iota = lax.iota(jnp.int32, LANES)
        dnums = lax.GatherDimensionNumbers(
            offset_dims=(), collapsed_slice_dims=(0,), start_index_map=(0,))

        def body(i_vmem, p_vmem, o_vmem):
            pltpu.sync_copy(tok_hbm.at[i_vmem.at[0]], o_vmem)

            @pl.loop(0, GATHER_WINDOW, step=LANES)
            def _(c0):
                pvec = p_vmem[0, pl.ds(c0, LANES)]
                for l0 in range(0, LANES, 4):
                    vals = []
                    for l in range(l0, l0 + 4):
                        pb = lax.gather(
                            pvec, jnp.full((LANES, 1), l, jnp.int32), dnums,
                            (1,), mode=lax.GatherScatterMode.PROMISE_IN_BOUNDS)
                        for j in range(H // LANES):
                            pr = plsc.load_gather(ptab_vmem, [pb, iota + j * LANES])
                            vals.append((l, j, pr))
                    for l, j, pr in vals:
                        plsc.addupdate(
                            o_vmem.at[c0 + l, pl.ds(j * LANES, LANES)], pr)

        pltpu.emit_pipeline(
            body,
            grid=(n // GATHER_WINDOW,),
            in_specs=[
                pl.BlockSpec((1, GATHER_WINDOW), lambda i: (0, i)),
                pl.BlockSpec((1, GATHER_WINDOW), lambda i: (0, i)),
            ],
            out_specs=[pl.BlockSpec((GATHER_WINDOW, H), lambda i: (i, 0))],
            core_axis_name=("core", "subcore"),
            dimension_semantics=(pltpu.PARALLEL,),
        )(i_hbm, p_hbm, o_hbm)

    return gather_kernel(token_table, pos_table, ids, pids)


def _tc_body(x_ref, wp_ref, u_ref, c_ref, o_ref):
    x = x_ref[...]
    mm = jnp.dot(x, wp_ref[...], preferred_element_type=jnp.float32)
    sx = jnp.sum(x, axis=1, keepdims=True)
    sx2 = jnp.sum(x * x, axis=1, keepdims=True)
    mu = sx * (1.0 / H)
    var = sx2 * (1.0 / H) - mu * mu
    rstd = lax.rsqrt(var + EPS)
    o_ref[...] = (mm - mu * u_ref[...]) * rstd + c_ref[...]


def _tc_ln_proj_chunk(x, wp, u2d, c2d, chunk, prev):
    steps = NC // TC_BLOCK
    k0 = chunk * steps
    common = [
        pl.BlockSpec((TC_BLOCK, H), lambda i: (i, 0)),
        pl.BlockSpec((H, H_ATTN), lambda i: (0, 0)),
        pl.BlockSpec((1, H_ATTN), lambda i: (0, 0)),
        pl.BlockSpec((1, H_ATTN), lambda i: (0, 0)),
    ]
    out_spec = pl.BlockSpec((TC_BLOCK, H_ATTN), lambda i, k0=k0: (k0 + i, 0))
    out_shape = jax.ShapeDtypeStruct((N, H_ATTN), jnp.float32)
    if prev is None:
        return pl.pallas_call(
            _tc_body, grid=(steps,), in_specs=common,
            out_specs=out_spec, out_shape=out_shape,
        )(x, wp, u2d, c2d)

    def body_alias(prev_ref, *refs):
        del prev_ref
        _tc_body(*refs)

    return pl.pallas_call(
        body_alias, grid=(steps,),
        in_specs=[pl.BlockSpec(memory_space=pl.ANY)] + common,
        out_specs=out_spec, out_shape=out_shape,
        input_output_aliases={0: 0},
    )(prev, x, wp, u2d, c2d)


def kernel(input, pos, token_table, pos_table, gamma, beta, W, b):
    ids = input.reshape(K_CHUNKS, 1, NC).astype(jnp.int32)
    pids = pos.reshape(K_CHUNKS, 1, NC).astype(jnp.int32)
    wt = W.T
    wp = wt * gamma.reshape(H, 1)
    u2d = jnp.sum(wp, axis=0).reshape(1, H_ATTN)
    c2d = (beta @ wt + b).reshape(1, H_ATTN)
    out = None
    for k in range(K_CHUNKS):
        x = _sc_gather_add(token_table, pos_table, ids[k], pids[k])
        out = _tc_ln_proj_chunk(x, wp, u2d, c2d, k, out)
    return out.reshape(B, L, H_ATTN)


# speedup vs baseline: 1.0746x; 1.0746x over previous
import dataclasses

import jax
import jax.numpy as jnp
from jax import lax
from jax.experimental import pallas as pl
from jax.experimental.pallas import tpu as pltpu
from jax.experimental.pallas import tpu_sc as plsc

B = 4096
L = 200
H = 128
H_ATTN = 128
MAX_LEN = 200
N = B * L
EPS = 1e-5

GATHER_WINDOW = 128
TC_BLOCK = 2048
K_CHUNKS = 4
NC = N // K_CHUNKS
LANES = 16


def _sc_gather_add(token_table, pos_table, ids, pids):
    n = ids.shape[1]
    mesh = plsc.VectorSubcoreMesh(core_axis_name="core", subcore_axis_name="subcore")

    cp = pltpu.CompilerParams()
    if "needs_layout_passes" in pltpu.CompilerParams.__dataclass_fields__:
        cp = dataclasses.replace(cp, needs_layout_passes=False)

    @pl.kernel(
        out_type=jax.ShapeDtypeStruct((n, H), jnp.float32),
        mesh=mesh,
        scratch_types=[
            pltpu.VMEM((MAX_LEN, H), jnp.float32),
            pltpu.SemaphoreType.DMA,
            pltpu.SemaphoreType.DMA,
        ],
        compiler_params=cp,
    )
    def gather_kernel(tok_hbm, ptab_hbm, i_hbm, p_hbm, o_hbm,
                      ptab_vmem, sem_a, sem_b):
        pltpu.sync_copy(ptab_hbm, ptab_vmem)
        iota = lax.iota(jnp.int32, LANES)
        dnums = lax.GatherDimensionNumbers(
            offset_dims=(), collapsed_slice_dims=(0,), start_index_map=(0,))
        W_ = GATHER_WINDOW

        def add_half(p_vmem, o_vmem, base):
            @pl.loop(0, W_, step=LANES)
            def _(c0):
                t0 = c0 + base
                pvec = p_vmem[0, pl.ds(t0, LANES)]
                for l0 in range(0, LANES, 4):
                    vals = []
                    for l in range(l0, l0 + 4):
                        pb = lax.gather(
                            pvec, jnp.full((LANES, 1), l, jnp.int32), dnums,
                            (1,), mode=lax.GatherScatterMode.PROMISE_IN_BOUNDS)
                        for j in range(H // LANES):
                            pr = plsc.load_gather(ptab_vmem, [pb, iota + j * LANES])
                            vals.append((l, j, pr))
                    for l, j, pr in vals:
                        plsc.addupdate(
                            o_vmem.at[t0 + l, pl.ds(j * LANES, LANES)], pr)

        def body(i_vmem, p_vmem, o_vmem):
            cpa = pltpu.async_copy(
                tok_hbm.at[i_vmem.at[0, pl.ds(0, W_)]],
                o_vmem.at[pl.ds(0, W_)], sem_a)
            cpb = pltpu.async_copy(
                tok_hbm.at[i_vmem.at[0, pl.ds(W_, W_)]],
                o_vmem.at[pl.ds(W_, W_)], sem_b)
            cpa.wait()
            add_half(p_vmem, o_vmem, 0)
            cpb.wait()
            add_half(p_vmem, o_vmem, W_)

        pltpu.emit_pipeline(
            body,
            grid=(n // (2 * W_),),
            in_specs=[
                pl.BlockSpec((1, 2 * W_), lambda i: (0, i)),
                pl.BlockSpec((1, 2 * W_), lambda i: (0, i)),
            ],
            out_specs=[pl.BlockSpec((2 * W_, H), lambda i: (i, 0))],
            core_axis_name=("core", "subcore"),
            dimension_semantics=(pltpu.PARALLEL,),
        )(i_hbm, p_hbm, o_hbm)

    return gather_kernel(token_table, pos_table, ids, pids)


def _tc_body(x_ref, wp_ref, u_ref, c_ref, o_ref):
    x = x_ref[...]
    mm = jnp.dot(x, wp_ref[...], preferred_element_type=jnp.float32)
    sx = jnp.sum(x, axis=1, keepdims=True)
    sx2 = jnp.sum(x * x, axis=1, keepdims=True)
    mu = sx * (1.0 / H)
    var = sx2 * (1.0 / H) - mu * mu
    rstd = lax.rsqrt(var + EPS)
    o_ref[...] = (mm - mu * u_ref[...]) * rstd + c_ref[...]


def _tc_ln_proj_chunk(x, wp, u2d, c2d, chunk, prev):
    steps = NC // TC_BLOCK
    k0 = chunk * steps
    common = [
        pl.BlockSpec((TC_BLOCK, H), lambda i: (i, 0)),
        pl.BlockSpec((H, H_ATTN), lambda i: (0, 0)),
        pl.BlockSpec((1, H_ATTN), lambda i: (0, 0)),
        pl.BlockSpec((1, H_ATTN), lambda i: (0, 0)),
    ]
    out_spec = pl.BlockSpec((TC_BLOCK, H_ATTN), lambda i, k0=k0: (k0 + i, 0))
    out_shape = jax.ShapeDtypeStruct((N, H_ATTN), jnp.float32)
    if prev is None:
        return pl.pallas_call(
            _tc_body, grid=(steps,), in_specs=common,
            out_specs=out_spec, out_shape=out_shape,
        )(x, wp, u2d, c2d)

    def body_alias(prev_ref, *refs):
        del prev_ref
        _tc_body(*refs)

    return pl.pallas_call(
        body_alias, grid=(steps,),
        in_specs=[pl.BlockSpec(memory_space=pl.ANY)] + common,
        out_specs=out_spec, out_shape=out_shape,
        input_output_aliases={0: 0},
    )(prev, x, wp, u2d, c2d)


def kernel(input, pos, token_table, pos_table, gamma, beta, W, b):
    ids = input.reshape(K_CHUNKS, 1, NC).astype(jnp.int32)
    pids = pos.reshape(K_CHUNKS, 1, NC).astype(jnp.int32)
    wt = W.T
    wp = wt * gamma.reshape(H, 1)
    u2d = jnp.sum(wp, axis=0).reshape(1, H_ATTN)
    c2d = (beta @ wt + b).reshape(1, H_ATTN)
    out = None
    for k in range(K_CHUNKS):
        x = _sc_gather_add(token_table, pos_table, ids[k], pids[k])
        out = _tc_ln_proj_chunk(x, wp, u2d, c2d, k, out)
    return out.reshape(B, L, H_ATTN)


# speedup vs baseline: 1.1033x; 1.0267x over previous
import dataclasses

import jax
import jax.numpy as jnp
from jax import lax
from jax.experimental import pallas as pl
from jax.experimental.pallas import tpu as pltpu
from jax.experimental.pallas import tpu_sc as plsc

B = 4096
L = 200
H = 128
H_ATTN = 128
MAX_LEN = 200
N = B * L
EPS = 1e-5

GATHER_WINDOW = 128
TC_BLOCK = 2048
K_CHUNKS = 8
NC = N // K_CHUNKS
LANES = 16


def _sc_gather_add(token_table, pos_table, ids, pids):
    n = ids.shape[1]
    mesh = plsc.VectorSubcoreMesh(core_axis_name="core", subcore_axis_name="subcore")

    cp = pltpu.CompilerParams()
    if "needs_layout_passes" in pltpu.CompilerParams.__dataclass_fields__:
        cp = dataclasses.replace(cp, needs_layout_passes=False)

    @pl.kernel(
        out_type=jax.ShapeDtypeStruct((n, H), jnp.float32),
        mesh=mesh,
        scratch_types=[
            pltpu.VMEM((MAX_LEN, H), jnp.float32),
            pltpu.SemaphoreType.DMA,
            pltpu.SemaphoreType.DMA,
        ],
        compiler_params=cp,
    )
    def gather_kernel(tok_hbm, ptab_hbm, i_hbm, p_hbm, o_hbm,
                      ptab_vmem, sem_a, sem_b):
        pltpu.sync_copy(ptab_hbm, ptab_vmem)
        iota = lax.iota(jnp.int32, LANES)
        dnums = lax.GatherDimensionNumbers(
            offset_dims=(), collapsed_slice_dims=(0,), start_index_map=(0,))
        W_ = GATHER_WINDOW

        def add_half(p_vmem, o_vmem, base):
            @pl.loop(0, W_, step=LANES)
            def _(c0):
                t0 = c0 + base
                pvec = p_vmem[0, pl.ds(t0, LANES)]
                for l0 in range(0, LANES, 4):
                    vals = []
                    for l in range(l0, l0 + 4):
                        pb = lax.gather(
                            pvec, jnp.full((LANES, 1), l, jnp.int32), dnums,
                            (1,), mode=lax.GatherScatterMode.PROMISE_IN_BOUNDS)
                        for j in range(H // LANES):
                            pr = plsc.load_gather(ptab_vmem, [pb, iota + j * LANES])
                            vals.append((l, j, pr))
                    for l, j, pr in vals:
                        plsc.addupdate(
                            o_vmem.at[t0 + l, pl.ds(j * LANES, LANES)], pr)

        def body(i_vmem, p_vmem, o_vmem):
            cpa = pltpu.async_copy(
                tok_hbm.at[i_vmem.at[0, pl.ds(0, W_)]],
                o_vmem.at[pl.ds(0, W_)], sem_a)
            cpb = pltpu.async_copy(
                tok_hbm.at[i_vmem.at[0, pl.ds(W_, W_)]],
                o_vmem.at[pl.ds(W_, W_)], sem_b)
            cpa.wait()
            add_half(p_vmem, o_vmem, 0)
            cpb.wait()
            add_half(p_vmem, o_vmem, W_)

        pltpu.emit_pipeline(
            body,
            grid=(n // (2 * W_),),
            in_specs=[
                pl.BlockSpec((1, 2 * W_), lambda i: (0, i)),
                pl.BlockSpec((1, 2 * W_), lambda i: (0, i)),
            ],
            out_specs=[pl.BlockSpec((2 * W_, H), lambda i: (i, 0))],
            core_axis_name=("core", "subcore"),
            dimension_semantics=(pltpu.PARALLEL,),
        )(i_hbm, p_hbm, o_hbm)

    return gather_kernel(token_table, pos_table, ids, pids)


def _tc_body(x_ref, wp_ref, u_ref, c_ref, o_ref):
    x = x_ref[...]
    mm = jnp.dot(x, wp_ref[...], preferred_element_type=jnp.float32)
    sx = jnp.sum(x, axis=1, keepdims=True)
    sx2 = jnp.sum(x * x, axis=1, keepdims=True)
    mu = sx * (1.0 / H)
    var = sx2 * (1.0 / H) - mu * mu
    rstd = lax.rsqrt(var + EPS)
    o_ref[...] = (mm - mu * u_ref[...]) * rstd + c_ref[...]


def _tc_ln_proj_chunk(x, wp, u2d, c2d, chunk, prev):
    steps = NC // TC_BLOCK
    k0 = chunk * steps
    common = [
        pl.BlockSpec((TC_BLOCK, H), lambda i: (i, 0)),
        pl.BlockSpec((H, H_ATTN), lambda i: (0, 0)),
        pl.BlockSpec((1, H_ATTN), lambda i: (0, 0)),
        pl.BlockSpec((1, H_ATTN), lambda i: (0, 0)),
    ]
    out_spec = pl.BlockSpec((TC_BLOCK, H_ATTN), lambda i, k0=k0: (k0 + i, 0))
    out_shape = jax.ShapeDtypeStruct((N, H_ATTN), jnp.float32)
    if prev is None:
        return pl.pallas_call(
            _tc_body, grid=(steps,), in_specs=common,
            out_specs=out_spec, out_shape=out_shape,
        )(x, wp, u2d, c2d)

    def body_alias(prev_ref, *refs):
        del prev_ref
        _tc_body(*refs)

    return pl.pallas_call(
        body_alias, grid=(steps,),
        in_specs=[pl.BlockSpec(memory_space=pl.ANY)] + common,
        out_specs=out_spec, out_shape=out_shape,
        input_output_aliases={0: 0},
    )(prev, x, wp, u2d, c2d)


def kernel(input, pos, token_table, pos_table, gamma, beta, W, b):
    ids = input.reshape(K_CHUNKS, 1, NC).astype(jnp.int32)
    pids = pos.reshape(K_CHUNKS, 1, NC).astype(jnp.int32)
    wt = W.T
    wp = wt * gamma.reshape(H, 1)
    u2d = jnp.sum(wp, axis=0).reshape(1, H_ATTN)
    c2d = (beta @ wt + b).reshape(1, H_ATTN)
    out = None
    for k in range(K_CHUNKS):
        x = _sc_gather_add(token_table, pos_table, ids[k], pids[k])
        out = _tc_ln_proj_chunk(x, wp, u2d, c2d, k, out)
    return out.reshape(B, L, H_ATTN)
